# baseline (speedup 1.0000x reference)
import jax
import jax.numpy as jnp
from jax.experimental import pallas as pl
from jax.experimental.pallas import tpu as pltpu

TOTAL = 32768
B = 16
IN_CH = 4
HID = 64
FEAT = 16
OUT = 256
P = 3.0
EPS = 1e-6

CHUNK = 32768
NUM = TOTAL // CHUNK


def _gem_kernel(featsT_ref, ids_ref, w1t_ref, w2t_ref, wfc_ref, out_ref,
                seg_ref):
    i = pl.program_id(0)

    @pl.when(i == 0)
    def _init():
        seg_ref[...] = jnp.zeros_like(seg_ref)

    x = jnp.maximum(
        jnp.dot(w1t_ref[...], featsT_ref[...],
                preferred_element_type=jnp.float32), 0.0)
    x = jnp.dot(w2t_ref[...], x,
                preferred_element_type=jnp.float32)
    xc = jnp.maximum(x, EPS)
    xp = xc * xc * xc
    xp_ext = jnp.pad(xp, ((0, 1), (0, 0)),
                     constant_values=1.0).astype(jnp.bfloat16)

    ids = ids_ref[0, 0, :]
    onehot = (ids[None, :] == jax.lax.broadcasted_iota(
        jnp.int32, (B, CHUNK), 0)).astype(jnp.bfloat16)
    seg_ref[...] += jax.lax.dot_general(
        onehot, xp_ext, dimension_numbers=(((1,), (1,)), ((), ())),
        preferred_element_type=jnp.float32)

    @pl.when(i == NUM - 1)
    def _finish():
        cnt = seg_ref[:, FEAT]
        max_len = jnp.max(cnt)
        pad = (max_len - cnt)[:, None] * (EPS ** 3)
        gem = jnp.power((seg_ref[:, :FEAT] + pad) / max_len, 1.0 / 3.0)
        out_ref[...] = jnp.dot(gem, wfc_ref[...],
                               preferred_element_type=jnp.float32)


@jax.jit
def kernel(feats, batch_ids, W1, W2, Wfc):
    ids3 = batch_ids.reshape(NUM, 1, CHUNK)
    featsT = feats.T
    return pl.pallas_call(
        _gem_kernel,
        grid=(NUM,),
        in_specs=[
            pl.BlockSpec((IN_CH, CHUNK), lambda i: (0, i)),
            pl.BlockSpec((1, 1, CHUNK), lambda i: (i, 0, 0)),
            pl.BlockSpec((HID, IN_CH), lambda i: (0, 0)),
            pl.BlockSpec((FEAT, HID), lambda i: (0, 0)),
            pl.BlockSpec((FEAT, OUT), lambda i: (0, 0)),
        ],
        out_specs=pl.BlockSpec((B, OUT), lambda i: (0, 0)),
        out_shape=jax.ShapeDtypeStruct((B, OUT), jnp.float32),
        scratch_shapes=[
            pltpu.VMEM((B, FEAT + 1), jnp.float32),
        ],
    )(featsT, ids3, W1.T, W2.T, Wfc)
